# baseline (speedup 1.0000x reference)
import functools

import jax
import jax.numpy as jnp
from jax import lax
from jax.experimental import pallas as pl
from jax.experimental.pallas import tpu as pltpu
from jax.experimental.pallas import tpu_sc as plsc

B = 16384
HID = 1024
NC, NS, L = 2, 16, 16
NW = NC * NS
CHUNK = 128
TROWS = 4120

NSLICE = 2
SB = B // NSLICE
BW = SB // NW
NJ = BW // CHUNK

TILE = 512
GRID = SB // TILE


@functools.cache
def _sc_gather_kernel():
    mesh = plsc.VectorSubcoreMesh(core_axis_name="c", subcore_axis_name="s")

    @functools.partial(
        pl.kernel,
        mesh=mesh,
        out_type=jax.ShapeDtypeStruct((SB, 128), jnp.float32),
        scratch_types=[
            pltpu.VMEM((NJ, CHUNK), jnp.int32),
            pltpu.VMEM((NJ, CHUNK), jnp.int32),
            pltpu.VMEM((NJ, CHUNK), jnp.int32),
            pltpu.VMEM((NJ, CHUNK), jnp.int32),
            pltpu.VMEM((NJ, CHUNK), jnp.int32),
            pltpu.VMEM((BW, 128), jnp.float32),
            pltpu.SemaphoreType.DMA,
        ],
    )
    def _sc_gather(i0_hbm, i1_hbm, i2_hbm, i3_hbm, table_hbm, x_hbm,
                   idx0_v, idx1_v, idx2_v, idx3_v, cidx_v, rows_v, sem):
        wid = lax.axis_index("s") * NC + lax.axis_index("c")
        pltpu.sync_copy(i0_hbm.at[wid], idx0_v)
        pltpu.sync_copy(i1_hbm.at[wid], idx1_v)
        pltpu.sync_copy(i2_hbm.at[wid], idx2_v)
        pltpu.sync_copy(i3_hbm.at[wid], idx3_v)
        for j in range(NJ):
            for k in range(CHUNK // L):
                s = pl.ds(k * L, L)
                cidx_v[j, s] = ((idx0_v[j, s] * 12 + idx1_v[j, s]) * 49
                                + idx2_v[j, s] * 7 + idx3_v[j, s])
        copies = []
        for j in range(NJ):
            copies.append(pltpu.async_copy(
                table_hbm.at[cidx_v.at[j]],
                rows_v.at[pl.ds(j * CHUNK, CHUNK)], sem))
        for c in copies:
            c.wait()
        pltpu.sync_copy(rows_v, x_hbm.at[pl.ds(wid * BW, BW)])

    return _sc_gather


def _mlp_body(*refs):
    x_ref, w1_ref, w2_ref, b2_ref = refs[:4]
    out_ref = refs[-1]
    h = jnp.dot(x_ref[...].astype(jnp.bfloat16), w1_ref[...],
                preferred_element_type=jnp.float32)
    h = h * (1.0 / (1.0 + jnp.exp(-h)))
    out = jnp.dot(h.astype(jnp.bfloat16), w2_ref[...],
                  preferred_element_type=jnp.float32)
    out_ref[...] = out + b2_ref[...]


def _mlp_call(x, w1p, w2, b2r, carry, slice_idx):
    full = lambda s: pl.BlockSpec(s, lambda i: (0, 0))
    off = slice_idx * GRID
    in_specs = [
        pl.BlockSpec((TILE, 128), lambda i: (i, 0)),
        full((128, HID)),
        full((HID, HID)),
        full((1, HID)),
    ]
    args = [x, w1p, w2, b2r]
    aliases = {}
    if carry is not None:
        in_specs.append(pl.BlockSpec(memory_space=pl.ANY))
        args.append(carry)
        aliases = {4: 0}
    return pl.pallas_call(
        _mlp_body,
        grid=(GRID,),
        in_specs=in_specs,
        out_specs=pl.BlockSpec((TILE, HID), lambda i, off=off: (i + off, 0)),
        out_shape=jax.ShapeDtypeStruct((B, HID), jnp.float32),
        input_output_aliases=aliases,
    )(*args)


def kernel(cal, dow_emb, month_emb, W1, b1, W2, b2):
    cal = cal.astype(jnp.int32)
    idx = cal.reshape(NSLICE, NW, NJ, CHUNK, 4)

    f = jnp.arange(7, dtype=jnp.float32)
    cd = jnp.broadcast_to(dow_emb[:, None, None, None, :], (7, 12, 7, 7, 16))
    cm = jnp.broadcast_to(month_emb[None, :, None, None, :], (7, 12, 7, 7, 16))
    c2 = jnp.broadcast_to(f[None, None, :, None, None], (7, 12, 7, 7, 1))
    c3 = jnp.broadcast_to(f[None, None, None, :, None], (7, 12, 7, 7, 1))
    ones = jnp.ones((7, 12, 7, 7, 1), jnp.float32)
    zeros = jnp.zeros((7, 12, 7, 7, 93), jnp.float32)
    table = jnp.concatenate([cd, cm, c2, c3, ones, zeros],
                            axis=-1).reshape(4116, 128)
    table = jnp.concatenate(
        [table, jnp.zeros((TROWS - 4116, 128), jnp.float32)], axis=0)

    sc = _sc_gather_kernel()
    xs = [sc(idx[k, ..., 0], idx[k, ..., 1], idx[k, ..., 2], idx[k, ..., 3],
             table) for k in range(NSLICE)]

    w1p = jnp.concatenate(
        [W1, b1.reshape(1, HID), jnp.zeros((128 - 35, HID), jnp.float32)],
        axis=0).astype(jnp.bfloat16)
    w2c = W2.astype(jnp.bfloat16)
    b2r = b2.reshape(1, HID)

    out = None
    for k in range(NSLICE):
        out = _mlp_call(xs[k], w1p, w2c, b2r, out, k)
    return out


# speedup vs baseline: 1.3296x; 1.3296x over previous
import functools

import jax
import jax.numpy as jnp
from jax import lax
from jax.experimental import pallas as pl
from jax.experimental.pallas import tpu as pltpu
from jax.experimental.pallas import tpu_sc as plsc

B = 16384
HID = 1024
NC, NS, L = 2, 16, 16
NW = NC * NS
CHUNK = 128
GROUPS = 84
GSTRIDE = 64
TROWS = GROUPS * GSTRIDE

NSLICE = 1
SB = B // NSLICE
BW = SB // NW
NJ = BW // CHUNK

TILE = 1024
GRID = SB // TILE


@functools.cache
def _sc_gather_kernel():
    mesh = plsc.VectorSubcoreMesh(core_axis_name="c", subcore_axis_name="s")

    @functools.partial(
        pl.kernel,
        mesh=mesh,
        out_type=jax.ShapeDtypeStruct((SB, 128), jnp.float32),
        scratch_types=[
            pltpu.VMEM((NJ, CHUNK), jnp.int32),
            pltpu.VMEM((NJ, CHUNK), jnp.int32),
            pltpu.VMEM((NJ, CHUNK), jnp.int32),
            pltpu.VMEM((NJ, CHUNK), jnp.int32),
            pltpu.VMEM((NJ, CHUNK), jnp.int32),
            pltpu.VMEM((BW, 128), jnp.float32),
            pltpu.SemaphoreType.DMA,
        ],
    )
    def _sc_gather(i0_hbm, i1_hbm, i2_hbm, i3_hbm, table_hbm, x_hbm,
                   idx0_v, idx1_v, idx2_v, idx3_v, cidx_v, rows_v, sem):
        wid = lax.axis_index("s") * NC + lax.axis_index("c")
        pltpu.sync_copy(i0_hbm.at[wid], idx0_v)
        pltpu.sync_copy(i1_hbm.at[wid], idx1_v)
        pltpu.sync_copy(i2_hbm.at[wid], idx2_v)
        pltpu.sync_copy(i3_hbm.at[wid], idx3_v)
        for j in range(NJ):
            for k in range(CHUNK // L):
                s = pl.ds(k * L, L)
                cidx_v[j, s] = ((idx0_v[j, s] * 12 + idx1_v[j, s]) * GSTRIDE
                                + idx2_v[j, s] * 8 + idx3_v[j, s])
        copies = []
        for j in range(NJ):
            copies.append(pltpu.async_copy(
                table_hbm.at[cidx_v.at[j]],
                rows_v.at[pl.ds(j * CHUNK, CHUNK)], sem))
        for c in copies:
            c.wait()
        pltpu.sync_copy(rows_v, x_hbm.at[pl.ds(wid * BW, BW)])

    return _sc_gather


def _table_body(dowp_ref, monthp_ref, out_ref):
    g = lax.broadcasted_iota(jnp.int32, (GROUPS, 1), 0)
    a = g // 12
    b = g - a * 12
    j16 = lax.broadcasted_iota(jnp.int32, (1, 16), 1)
    oh_a = (a == j16).astype(jnp.bfloat16)
    oh_b = (b == j16).astype(jnp.bfloat16)
    base = (jnp.dot(oh_a, dowp_ref[...], preferred_element_type=jnp.float32)
            + jnp.dot(oh_b, monthp_ref[...],
                      preferred_element_type=jnp.float32))
    q = lax.broadcasted_iota(jnp.int32, (GSTRIDE, 1), 0)
    f2 = (q >> 3).astype(jnp.float32)
    f3 = (q & 7).astype(jnp.float32)
    c = lax.broadcasted_iota(jnp.int32, (GSTRIDE, 128), 1)
    pat = (jnp.where(c == 32, f2, 0.0)
           + jnp.where(c == 33, f3, 0.0)
           + jnp.where(c == 34, 1.0, 0.0))
    for gg in range(GROUPS):
        out_ref[pl.ds(gg * GSTRIDE, GSTRIDE), :] = base[gg:gg + 1, :] + pat


def _table_call(dowp, monthp):
    full = lambda s: pl.BlockSpec(s, lambda: (0, 0))
    return pl.pallas_call(
        _table_body,
        in_specs=[full((16, 128)), full((16, 128))],
        out_specs=full((TROWS, 128)),
        out_shape=jax.ShapeDtypeStruct((TROWS, 128), jnp.float32),
    )(dowp, monthp)


def _mlp_body(*refs):
    x_ref, w1_ref, w2_ref, b2_ref = refs[:4]
    out_ref = refs[-1]
    h = jnp.dot(x_ref[...].astype(jnp.bfloat16), w1_ref[...],
                preferred_element_type=jnp.float32)
    h = 0.5 * h * (1.0 + lax.tanh(0.5 * h))
    out = jnp.dot(h.astype(jnp.bfloat16), w2_ref[...],
                  preferred_element_type=jnp.float32)
    out_ref[...] = out + b2_ref[...]


def _mlp_call(x, w1p, w2, b2r, carry, slice_idx):
    full = lambda s: pl.BlockSpec(s, lambda i: (0, 0))
    off = slice_idx * GRID
    in_specs = [
        pl.BlockSpec((TILE, 128), lambda i: (i, 0)),
        full((128, HID)),
        full((HID, HID)),
        full((1, HID)),
    ]
    args = [x, w1p, w2, b2r]
    aliases = {}
    if carry is not None:
        in_specs.append(pl.BlockSpec(memory_space=pl.ANY))
        args.append(carry)
        aliases = {4: 0}
    return pl.pallas_call(
        _mlp_body,
        grid=(GRID,),
        in_specs=in_specs,
        out_specs=pl.BlockSpec((TILE, HID), lambda i, off=off: (i + off, 0)),
        out_shape=jax.ShapeDtypeStruct((B, HID), jnp.float32),
        input_output_aliases=aliases,
    )(*args)


def kernel(cal, dow_emb, month_emb, W1, b1, W2, b2):
    cal = cal.astype(jnp.int32)
    idx = cal.reshape(NSLICE, NW, NJ, CHUNK, 4)

    dowp = jnp.pad(dow_emb, ((0, 9), (0, 112))).astype(jnp.bfloat16)
    monthp = jnp.pad(month_emb, ((0, 4), (16, 96))).astype(jnp.bfloat16)
    table = _table_call(dowp, monthp)

    sc = _sc_gather_kernel()
    xs = [sc(idx[k, ..., 0], idx[k, ..., 1], idx[k, ..., 2], idx[k, ..., 3],
             table) for k in range(NSLICE)]

    w1p = jnp.concatenate(
        [W1, b1.reshape(1, HID), jnp.zeros((128 - 35, HID), jnp.float32)],
        axis=0).astype(jnp.bfloat16)
    w2c = W2.astype(jnp.bfloat16)
    b2r = b2.reshape(1, HID)

    out = None
    for k in range(NSLICE):
        out = _mlp_call(xs[k], w1p, w2c, b2r, out, k)
    return out
